# baseline (speedup 1.0000x reference)
import functools

import jax
import jax.numpy as jnp
from jax import lax
from jax.experimental import pallas as pl
from jax.experimental.pallas import tpu as pltpu
from jax.experimental.pallas import tpu_sc as plsc

VOCAB = 1000000
D = 64
B = 4096
SEQ = 200
EPS = 1e-5

NC, NS = 2, 16
NW = NC * NS
BB = B // NW
NLT = SEQ // 8
NITEM = SEQ

_mesh = plsc.VectorSubcoreMesh(core_axis_name="c", subcore_axis_name="s")


def _rsqrt(x):
    i = lax.bitcast_convert_type(x, jnp.int32)
    i = jnp.int32(0x5F3759DF) - lax.shift_right_logical(i, 1)
    y = lax.bitcast_convert_type(i, jnp.float32)
    hx = 0.5 * x
    y = y * (1.5 - hx * y * y)
    y = y * (1.5 - hx * y * y)
    y = y * (1.5 - hx * y * y)
    return y


@functools.partial(
    pl.kernel,
    out_type=jax.ShapeDtypeStruct((SEQ, D // 8, NW, 8, BB), jnp.float32),
    mesh=_mesh,
    scratch_types=[
        pltpu.VMEM((2, 8, BB), jnp.int32),
        pltpu.VMEM((2, BB, D), jnp.float32),
        pltpu.VMEM((2, D, BB), jnp.float32),
        pltpu.VMEM((2, 2, 8, 16), jnp.float32),
        pltpu.VMEM((D,), jnp.float32),
        pltpu.VMEM((D,), jnp.float32),
        pltpu.VMEM((D, 16), jnp.float32),
        pltpu.VMEM((D, 16), jnp.float32),
        pltpu.SemaphoreType.DMA((2,)),
        pltpu.SemaphoreType.DMA((2,)),
    ],
    compiler_params=pltpu.CompilerParams(
        needs_layout_passes=False, use_tc_tiling_on_sc=False),
)
def _emb_ln(x4_hbm, table_hbm, gamma_hbm, beta_hbm, out_hbm,
            idx_v, rows_v, obuf_v, stats_v, gamma_v, beta_v,
            gspl, bspl, gsem, osem):
    w = lax.axis_index("s") * NC + lax.axis_index("c")

    pltpu.sync_copy(gamma_hbm, gamma_v)
    pltpu.sync_copy(beta_hbm, beta_v)
    iota16 = lax.iota(jnp.int32, 16)
    zeros16 = jnp.zeros((16,), jnp.float32)

    for c in range(D // 16):
        gv = gamma_v[pl.ds(c * 16, 16)]
        bv = beta_v[pl.ds(c * 16, 16)]
        for j in range(16):
            d = c * 16 + j
            gspl[d] = jnp.full((16,), gv[j], jnp.float32)
            bspl[d] = jnp.full((16,), bv[j], jnp.float32)

    pltpu.sync_copy(x4_hbm.at[0, w], idx_v.at[0])
    pltpu.async_copy(table_hbm.at[idx_v.at[0, 0]], rows_v.at[0], gsem.at[0])

    def item_body(l, carry):
        cur = lax.rem(l, 2)
        nxt = lax.rem(l + 1, 2)
        lt = lax.div(l, 8)
        l8 = lax.rem(l, 8)

        @pl.when(jnp.logical_and(l8 == 0, lt + 1 < NLT))
        def _():
            pltpu.sync_copy(x4_hbm.at[lt + 1, w], idx_v.at[lax.rem(lt + 1, 2)])

        @pl.when(l + 1 < NITEM)
        def _():
            nlt = lax.div(l + 1, 8)
            nl8 = lax.rem(l + 1, 8)
            pltpu.async_copy(
                table_hbm.at[idx_v.at[lax.rem(nlt, 2), nl8]],
                rows_v.at[nxt], gsem.at[nxt])

        pltpu.make_async_copy(table_hbm.at[idx_v.at[0, 0]],
                              rows_v.at[cur], gsem.at[cur]).wait()

        @pl.when(l >= 2)
        def _():
            for dt in range(D // 8):
                pltpu.make_async_copy(
                    obuf_v.at[cur, pl.ds(dt * 8, 8)],
                    out_hbm.at[0, dt, 0], osem.at[cur]).wait()

        rows = rows_v.at[cur]
        obuf = obuf_v.at[cur]
        stats = stats_v.at[cur]

        def grp_body(g, c2):
            row16 = g * 16 + iota16

            @plsc.parallel_loop(0, D, unroll=16, carry=(zeros16, zeros16))
            def sq_loop(k, sq):
                s, q = sq
                cm = jnp.bitwise_and(iota16 + jnp.broadcast_to(k, (16,)),
                                     D - 1)
                dk = plsc.load_gather(rows, [row16, cm])
                plsc.store_scatter(obuf, [cm, row16], dk)
                return (s + dk, q + dk * dk)

            s, q = sq_loop
            mean = s * (1.0 / D)
            var = q * (1.0 / D) - mean * mean
            inv = _rsqrt(var + EPS)
            stats[0, g] = mean
            stats[1, g] = inv
            return c2

        lax.fori_loop(0, BB // 16, grp_body, 0)

        means = [stats[0, c] for c in range(8)]
        invs = [stats[1, c] for c in range(8)]

        @plsc.parallel_loop(0, D, unroll=8)
        def d_loop(d):
            gd = gspl[d]
            bd = bspl[d]
            for c in range(8):
                v = obuf[d, pl.ds(c * 16, 16)]
                obuf[d, pl.ds(c * 16, 16)] = (
                    (v - means[c]) * invs[c] * gd + bd)

        for dt in range(D // 8):
            pltpu.async_copy(obuf.at[pl.ds(dt * 8, 8)],
                             out_hbm.at[l, dt, w], osem.at[cur])
        return carry

    lax.fori_loop(0, NITEM, item_body, 0)

    for par in range(2):
        for dt in range(D // 8):
            pltpu.make_async_copy(
                obuf_v.at[par, pl.ds(dt * 8, 8)],
                out_hbm.at[0, dt, 0], osem.at[par]).wait()


def kernel(x, table, gamma, beta):
    x4 = x.astype(jnp.int32).T.reshape(NLT, 8, NW, BB).transpose(0, 2, 1, 3)
    o5 = _emb_ln(x4, table, gamma, beta)
    return o5.transpose(2, 4, 0, 1, 3).reshape(B, SEQ, D)


# speedup vs baseline: 1.1549x; 1.1549x over previous
import functools

import jax
import jax.numpy as jnp
from jax import lax
from jax.experimental import pallas as pl
from jax.experimental.pallas import tpu as pltpu
from jax.experimental.pallas import tpu_sc as plsc

VOCAB = 1000000
D = 64
B = 4096
SEQ = 200
EPS = 1e-5

NC, NS = 2, 16
NW = NC * NS
BB = B // NW
NLT = SEQ // 8
NITEM = SEQ

_mesh = plsc.VectorSubcoreMesh(core_axis_name="c", subcore_axis_name="s")


def _rsqrt(x):
    i = lax.bitcast_convert_type(x, jnp.int32)
    i = jnp.int32(0x5F3759DF) - lax.shift_right_logical(i, 1)
    y = lax.bitcast_convert_type(i, jnp.float32)
    hx = 0.5 * x
    y = y * (1.5 - hx * y * y)
    y = y * (1.5 - hx * y * y)
    y = y * (1.5 - hx * y * y)
    return y


@functools.partial(
    pl.kernel,
    out_type=jax.ShapeDtypeStruct((SEQ, D // 8, NW, 8, BB), jnp.float32),
    mesh=_mesh,
    scratch_types=[
        pltpu.VMEM((2, 8, BB), jnp.int32),
        pltpu.VMEM((2, BB, D), jnp.float32),
        pltpu.VMEM((2, D, BB), jnp.float32),
        pltpu.VMEM((2, 2, 8, 16), jnp.float32),
        pltpu.VMEM((D,), jnp.float32),
        pltpu.VMEM((D,), jnp.float32),
        pltpu.VMEM((D, 16), jnp.float32),
        pltpu.VMEM((D, 16), jnp.float32),
        pltpu.SemaphoreType.DMA((2,)),
        pltpu.SemaphoreType.DMA((2,)),
    ],
    compiler_params=pltpu.CompilerParams(
        needs_layout_passes=False, use_tc_tiling_on_sc=False),
)
def _emb_ln(x4_hbm, table_hbm, gamma_hbm, beta_hbm, out_hbm,
            idx_v, rows_v, obuf_v, stats_v, gamma_v, beta_v,
            gspl, bspl, gsem, osem):
    w = lax.axis_index("s") * NC + lax.axis_index("c")

    pltpu.sync_copy(gamma_hbm, gamma_v)
    pltpu.sync_copy(beta_hbm, beta_v)
    iota16 = lax.iota(jnp.int32, 16)
    zeros16 = jnp.zeros((16,), jnp.float32)

    for c in range(D // 16):
        gv = gamma_v[pl.ds(c * 16, 16)]
        bv = beta_v[pl.ds(c * 16, 16)]
        for j in range(16):
            d = c * 16 + j
            gspl[d] = jnp.full((16,), gv[j], jnp.float32)
            bspl[d] = jnp.full((16,), bv[j], jnp.float32)

    pltpu.sync_copy(x4_hbm.at[0, w], idx_v.at[0])
    pltpu.async_copy(table_hbm.at[idx_v.at[0, 0]], rows_v.at[0], gsem.at[0])

    def item_body(l, carry):
        cur = lax.rem(l, 2)
        nxt = lax.rem(l + 1, 2)
        lt = lax.div(l, 8)
        l8 = lax.rem(l, 8)

        @pl.when(jnp.logical_and(l8 == 0, lt + 1 < NLT))
        def _():
            pltpu.sync_copy(x4_hbm.at[lt + 1, w], idx_v.at[lax.rem(lt + 1, 2)])

        @pl.when(l + 1 < NITEM)
        def _():
            nlt = lax.div(l + 1, 8)
            nl8 = lax.rem(l + 1, 8)
            pltpu.async_copy(
                table_hbm.at[idx_v.at[lax.rem(nlt, 2), nl8]],
                rows_v.at[nxt], gsem.at[nxt])

        pltpu.make_async_copy(table_hbm.at[idx_v.at[0, 0]],
                              rows_v.at[cur], gsem.at[cur]).wait()

        @pl.when(l >= 2)
        def _():
            for dt in range(D // 8):
                pltpu.make_async_copy(
                    obuf_v.at[cur, pl.ds(dt * 8, 8)],
                    out_hbm.at[0, dt, 0], osem.at[cur]).wait()

        rows = rows_v.at[cur]
        obuf = obuf_v.at[cur]
        stats = stats_v.at[cur]

        def grp_body(g, c2):
            row16 = g * 16 + iota16

            @plsc.parallel_loop(0, D, unroll=32, carry=(zeros16, zeros16))
            def sq_loop(k, sq):
                s, q = sq
                cm = jnp.bitwise_and(iota16 + jnp.broadcast_to(k, (16,)),
                                     D - 1)
                dk = plsc.load_gather(rows, [row16, cm])
                plsc.store_scatter(obuf, [cm, row16], dk)
                return (s + dk, q + dk * dk)

            s, q = sq_loop
            mean = s * (1.0 / D)
            var = q * (1.0 / D) - mean * mean
            inv = _rsqrt(var + EPS)
            stats[0, g] = mean
            stats[1, g] = inv
            return c2

        lax.fori_loop(0, BB // 16, grp_body, 0)

        means = [stats[0, c] for c in range(8)]
        invs = [stats[1, c] for c in range(8)]

        @plsc.parallel_loop(0, D, unroll=4)
        def d_loop(d):
            gd = gspl[d]
            bd = bspl[d]
            for c in range(8):
                v = obuf[d, pl.ds(c * 16, 16)]
                obuf[d, pl.ds(c * 16, 16)] = (
                    (v - means[c]) * invs[c] * gd + bd)

        for dt in range(D // 8):
            pltpu.async_copy(obuf.at[pl.ds(dt * 8, 8)],
                             out_hbm.at[l, dt, w], osem.at[cur])
        return carry

    lax.fori_loop(0, NITEM, item_body, 0)

    for par in range(2):
        for dt in range(D // 8):
            pltpu.make_async_copy(
                obuf_v.at[par, pl.ds(dt * 8, 8)],
                out_hbm.at[0, dt, 0], osem.at[par]).wait()


def kernel(x, table, gamma, beta):
    x4 = x.astype(jnp.int32).T.reshape(NLT, 8, NW, BB).transpose(0, 2, 1, 3)
    o5 = _emb_ln(x4, table, gamma, beta)
    return o5.transpose(2, 4, 0, 1, 3).reshape(B, SEQ, D)


# speedup vs baseline: 1.1554x; 1.0004x over previous
import functools

import jax
import jax.numpy as jnp
from jax import lax
from jax.experimental import pallas as pl
from jax.experimental.pallas import tpu as pltpu
from jax.experimental.pallas import tpu_sc as plsc

VOCAB = 1000000
D = 64
B = 4096
SEQ = 200
EPS = 1e-5

NC, NS = 2, 16
NW = NC * NS
BB = B // NW
NLT = SEQ // 8
NITEM = SEQ

_mesh = plsc.VectorSubcoreMesh(core_axis_name="c", subcore_axis_name="s")


def _rsqrt(x):
    i = lax.bitcast_convert_type(x, jnp.int32)
    i = jnp.int32(0x5F3759DF) - lax.shift_right_logical(i, 1)
    y = lax.bitcast_convert_type(i, jnp.float32)
    hx = 0.5 * x
    y = y * (1.5 - hx * y * y)
    y = y * (1.5 - hx * y * y)
    y = y * (1.5 - hx * y * y)
    return y


@functools.partial(
    pl.kernel,
    out_type=jax.ShapeDtypeStruct((SEQ, D // 8, NW, 8, BB), jnp.float32),
    mesh=_mesh,
    scratch_types=[
        pltpu.VMEM((2, 8, BB), jnp.int32),
        pltpu.VMEM((3, BB, D), jnp.float32),
        pltpu.VMEM((2, D, BB), jnp.float32),
        pltpu.VMEM((2, 2, 8, 16), jnp.float32),
        pltpu.VMEM((D,), jnp.float32),
        pltpu.VMEM((D,), jnp.float32),
        pltpu.VMEM((D, 16), jnp.float32),
        pltpu.VMEM((D, 16), jnp.float32),
        pltpu.SemaphoreType.DMA((3,)),
        pltpu.SemaphoreType.DMA((2,)),
    ],
    compiler_params=pltpu.CompilerParams(
        needs_layout_passes=False, use_tc_tiling_on_sc=False),
)
def _emb_ln(x4_hbm, table_hbm, gamma_hbm, beta_hbm, out_hbm,
            idx_v, rows_v, obuf_v, stats_v, gamma_v, beta_v,
            gspl, bspl, gsem, osem):
    w = lax.axis_index("s") * NC + lax.axis_index("c")

    pltpu.sync_copy(gamma_hbm, gamma_v)
    pltpu.sync_copy(beta_hbm, beta_v)
    iota16 = lax.iota(jnp.int32, 16)
    zeros16 = jnp.zeros((16,), jnp.float32)

    for c in range(D // 16):
        gv = gamma_v[pl.ds(c * 16, 16)]
        bv = beta_v[pl.ds(c * 16, 16)]
        for j in range(16):
            d = c * 16 + j
            gspl[d] = jnp.full((16,), gv[j], jnp.float32)
            bspl[d] = jnp.full((16,), bv[j], jnp.float32)

    pltpu.sync_copy(x4_hbm.at[0, w], idx_v.at[0])
    pltpu.async_copy(table_hbm.at[idx_v.at[0, 0]], rows_v.at[0], gsem.at[0])
    pltpu.async_copy(table_hbm.at[idx_v.at[0, 1]], rows_v.at[1], gsem.at[1])

    def item_body(l, carry):
        cur = lax.rem(l, 3)
        cur2 = lax.rem(l, 2)
        lt = lax.div(l, 8)
        l8 = lax.rem(l, 8)

        @pl.when(jnp.logical_and(l8 == 0, lt + 1 < NLT))
        def _():
            pltpu.sync_copy(x4_hbm.at[lt + 1, w], idx_v.at[lax.rem(lt + 1, 2)])

        @pl.when(l + 2 < NITEM)
        def _():
            nlt = lax.div(l + 2, 8)
            nl8 = lax.rem(l + 2, 8)
            pltpu.async_copy(
                table_hbm.at[idx_v.at[lax.rem(nlt, 2), nl8]],
                rows_v.at[lax.rem(l + 2, 3)], gsem.at[lax.rem(l + 2, 3)])

        pltpu.make_async_copy(table_hbm.at[idx_v.at[0, 0]],
                              rows_v.at[cur], gsem.at[cur]).wait()

        @pl.when(l >= 2)
        def _():
            for dt in range(D // 8):
                pltpu.make_async_copy(
                    obuf_v.at[cur2, pl.ds(dt * 8, 8)],
                    out_hbm.at[0, dt, 0], osem.at[cur2]).wait()

        rows = rows_v.at[cur]
        obuf = obuf_v.at[cur2]
        stats = stats_v.at[cur2]

        def grp_body(g, c2):
            row16 = g * 16 + iota16

            @plsc.parallel_loop(0, D, unroll=32, carry=(zeros16, zeros16))
            def sq_loop(k, sq):
                s, q = sq
                cm = jnp.bitwise_and(iota16 + jnp.broadcast_to(k, (16,)),
                                     D - 1)
                dk = plsc.load_gather(rows, [row16, cm])
                plsc.store_scatter(obuf, [cm, row16], dk)
                return (s + dk, q + dk * dk)

            s, q = sq_loop
            mean = s * (1.0 / D)
            var = q * (1.0 / D) - mean * mean
            inv = _rsqrt(var + EPS)
            stats[0, g] = mean
            stats[1, g] = inv
            return c2

        lax.fori_loop(0, BB // 16, grp_body, 0)

        means = [stats[0, c] for c in range(8)]
        invs = [stats[1, c] for c in range(8)]

        @plsc.parallel_loop(0, D, unroll=4)
        def d_loop(d):
            gd = gspl[d]
            bd = bspl[d]
            for c in range(8):
                v = obuf[d, pl.ds(c * 16, 16)]
                obuf[d, pl.ds(c * 16, 16)] = (
                    (v - means[c]) * invs[c] * gd + bd)

        for dt in range(D // 8):
            pltpu.async_copy(obuf.at[pl.ds(dt * 8, 8)],
                             out_hbm.at[l, dt, w], osem.at[cur2])
        return carry

    lax.fori_loop(0, NITEM, item_body, 0)

    for par in range(2):
        for dt in range(D // 8):
            pltpu.make_async_copy(
                obuf_v.at[par, pl.ds(dt * 8, 8)],
                out_hbm.at[0, dt, 0], osem.at[par]).wait()


def kernel(x, table, gamma, beta):
    x4 = x.astype(jnp.int32).T.reshape(NLT, 8, NW, BB).transpose(0, 2, 1, 3)
    o5 = _emb_ln(x4, table, gamma, beta)
    return o5.transpose(2, 4, 0, 1, 3).reshape(B, SEQ, D)
